# Initial kernel scaffold; baseline (speedup 1.0000x reference)
import functools

import jax
import jax.numpy as jnp
from jax import lax
from jax.experimental import pallas as pl
from jax.experimental.pallas import tpu as pltpu
from jax.experimental.pallas import tpu_sc as plsc

N = 320000
D = 128
S = 10000
SP = 10240
NC, NS = 2, 16
ROWS_PER_TILE = N // (NC * NS)
CHUNK = 400
SUB = 80
NSUB = CHUNK // SUB
NCHUNK = ROWS_PER_TILE // CHUNK
SEG_PER_TILE = SP // NS
CPY = 320


def _sc_body(x_hbm, idx_hbm, zacc_hbm, zcnt_hbm, ones_hbm,
             sums_hbm, cnts_hbm,
             acc_sh, cnt_sh, chunk_v, idx_v, ones_v, stage_v, cstage_v):
    c = lax.axis_index("c")
    s = lax.axis_index("s")
    w = c * NS + s

    pltpu.sync_copy(zacc_hbm, stage_v)
    for k in range(SEG_PER_TILE // CPY):
        pltpu.sync_copy(stage_v, acc_sh.at[pl.ds(s * SEG_PER_TILE + k * CPY, CPY)])
    pltpu.sync_copy(zcnt_hbm, cstage_v)
    pltpu.sync_copy(cstage_v, cnt_sh.at[pl.ds(s * SEG_PER_TILE, SEG_PER_TILE)])
    pltpu.sync_copy(ones_hbm, ones_v)
    plsc.subcore_barrier()

    base = w * ROWS_PER_TILE
    ibase = w * (ROWS_PER_TILE // SUB)

    def body(i, carry):
        r0 = base + i * CHUNK
        pltpu.sync_copy(x_hbm.at[pl.ds(r0, CHUNK)], chunk_v)
        pltpu.sync_copy(idx_hbm.at[pl.ds(ibase + i * NSUB, NSUB)], idx_v)
        for j in range(NSUB):
            pltpu.sync_copy(chunk_v.at[pl.ds(j * SUB, SUB)],
                            acc_sh.at[idx_v.at[j]], add=True)
            pltpu.sync_copy(ones_v, cnt_sh.at[idx_v.at[j]], add=True)
        return carry

    lax.fori_loop(0, NCHUNK, body, 0)
    plsc.subcore_barrier()

    for k in range(SEG_PER_TILE // CPY):
        off = s * SEG_PER_TILE + k * CPY
        pltpu.sync_copy(acc_sh.at[pl.ds(off, CPY)], stage_v)
        pltpu.sync_copy(stage_v, sums_hbm.at[c, pl.ds(off, CPY)])
    pltpu.sync_copy(cnt_sh.at[pl.ds(s * SEG_PER_TILE, SEG_PER_TILE)], cstage_v)
    pltpu.sync_copy(cstage_v, cnts_hbm.at[c, pl.ds(s * SEG_PER_TILE, SEG_PER_TILE)])


_sc_segment_sum = functools.partial(
    pl.kernel,
    out_type=(
        jax.ShapeDtypeStruct((NC, SP, D), jnp.float32),
        jax.ShapeDtypeStruct((NC, SP, 16), jnp.float32),
    ),
    mesh=plsc.VectorSubcoreMesh(core_axis_name="c", subcore_axis_name="s"),
    scratch_types=[
        pltpu.VMEM_SHARED((SP, D), jnp.float32),
        pltpu.VMEM_SHARED((SP, 16), jnp.float32),
        pltpu.VMEM((CHUNK, D), jnp.float32),
        pltpu.VMEM((NSUB, SUB), jnp.int32),
        pltpu.VMEM((SUB, 16), jnp.float32),
        pltpu.VMEM((CPY, D), jnp.float32),
        pltpu.VMEM((SEG_PER_TILE, 16), jnp.float32),
    ],
)(_sc_body)


def _tc_body(sums_ref, cnt_ref, w_ref, b_ref, o_ref):
    sums = sums_ref[0] + sums_ref[1]
    cnt = (jnp.max(cnt_ref[0], axis=-1, keepdims=True)
           + jnp.max(cnt_ref[1], axis=-1, keepdims=True))
    mean = sums / jnp.maximum(cnt, 1.0)
    out = lax.dot_general(mean, w_ref[...], (((1,), (1,)), ((), ())),
                          preferred_element_type=jnp.float32)
    o_ref[...] = out + jnp.where(cnt > 0.0, b_ref[...], 0.0)


def kernel(x, batch, W, b):
    batch = batch.astype(jnp.int32).reshape(N // SUB, SUB)
    zacc = jnp.zeros((CPY, D), jnp.float32)
    zcnt = jnp.zeros((SEG_PER_TILE, 16), jnp.float32)
    ones = jnp.ones((SUB, 16), jnp.float32)
    sums_p, cnts_p = _sc_segment_sum(x, batch, zacc, zcnt, ones)
    out = pl.pallas_call(
        _tc_body,
        out_shape=jax.ShapeDtypeStruct((SP, D), jnp.float32),
    )(sums_p, cnts_p, W, b.reshape(1, D))
    return out[:S]


# speedup vs baseline: 5.7642x; 5.7642x over previous
import functools

import jax
import jax.numpy as jnp
from jax import lax
from jax.experimental import pallas as pl
from jax.experimental.pallas import tpu as pltpu
from jax.experimental.pallas import tpu_sc as plsc

N = 320000
D = 128
S = 10000
SP = 10240
NC, NS = 2, 16
NW = NC * NS
CHUNK = 128
TOT_CHUNKS = N // CHUNK
ITERS = -(-TOT_CHUNKS // NW)
SEG_PER_TILE = SP // NS


def _sc_body(x_hbm, idx_hbm, zacc_hbm, zcnt_hbm, ones_hbm,
             sums_hbm, cnts_hbm,
             acc_sh, cnt_sh, chunk_v, idx_v, ones_v, cstage_v):
    c = lax.axis_index("c")
    s = lax.axis_index("s")
    w = c * NS + s

    pltpu.sync_copy(zacc_hbm, chunk_v)
    pltpu.sync_copy(zcnt_hbm, cstage_v)
    for k in range(SEG_PER_TILE // CHUNK):
        off = s * SEG_PER_TILE + k * CHUNK
        pltpu.sync_copy(chunk_v, acc_sh.at[pl.ds(off, CHUNK)])
    pltpu.sync_copy(cstage_v, cnt_sh.at[pl.ds(s * SEG_PER_TILE, SEG_PER_TILE)])
    pltpu.sync_copy(ones_hbm, ones_v)
    plsc.subcore_barrier()

    def body(i, carry):
        cid = i * NW + w

        @pl.when(cid < TOT_CHUNKS)
        def _():
            pltpu.sync_copy(x_hbm.at[cid], chunk_v)
            pltpu.sync_copy(idx_hbm.at[cid], idx_v)
            pltpu.sync_copy(chunk_v, acc_sh.at[idx_v.at[0]], add=True)
            pltpu.sync_copy(ones_v, cnt_sh.at[idx_v.at[0]], add=True)

        return carry

    lax.fori_loop(0, ITERS, body, 0)
    plsc.subcore_barrier()

    for k in range(SEG_PER_TILE // CHUNK):
        off = s * SEG_PER_TILE + k * CHUNK
        pltpu.sync_copy(acc_sh.at[pl.ds(off, CHUNK)], chunk_v)
        pltpu.sync_copy(chunk_v, sums_hbm.at[c, pl.ds(off, CHUNK)])
    pltpu.sync_copy(cnt_sh.at[pl.ds(s * SEG_PER_TILE, SEG_PER_TILE)], cstage_v)
    pltpu.sync_copy(cstage_v, cnts_hbm.at[c, pl.ds(s * SEG_PER_TILE, SEG_PER_TILE)])


_sc_segment_sum = functools.partial(
    pl.kernel,
    out_type=(
        jax.ShapeDtypeStruct((NC, SP, D), jnp.float32),
        jax.ShapeDtypeStruct((NC, SP), jnp.float32),
    ),
    mesh=plsc.VectorSubcoreMesh(core_axis_name="c", subcore_axis_name="s"),
    scratch_types=[
        pltpu.VMEM_SHARED((SP, D), jnp.float32),
        pltpu.VMEM_SHARED((SP,), jnp.float32),
        pltpu.VMEM((CHUNK, D), jnp.float32),
        pltpu.VMEM((1, CHUNK), jnp.int32),
        pltpu.VMEM((CHUNK,), jnp.float32),
        pltpu.VMEM((SEG_PER_TILE,), jnp.float32),
    ],
)(_sc_body)


def _tc_body(sums_ref, cnt_ref, w_ref, b_ref, o_ref):
    sums = sums_ref[0] + sums_ref[1]
    cnt = cnt_ref[0] + cnt_ref[1]
    mean = sums / jnp.maximum(cnt, 1.0)
    out = lax.dot_general(mean, w_ref[...], (((1,), (1,)), ((), ())),
                          preferred_element_type=jnp.float32)
    o_ref[...] = out + jnp.where(cnt > 0.0, b_ref[...], 0.0)


def kernel(x, batch, W, b):
    x3 = x.reshape(TOT_CHUNKS, CHUNK, D)
    batch3 = batch.astype(jnp.int32).reshape(TOT_CHUNKS, 1, CHUNK)
    zacc = jnp.zeros((CHUNK, D), jnp.float32)
    zcnt = jnp.zeros((SEG_PER_TILE,), jnp.float32)
    ones = jnp.ones((CHUNK,), jnp.float32)
    sums_p, cnts_p = _sc_segment_sum(x3, batch3, zacc, zcnt, ones)
    cnts_p = cnts_p.reshape(NC, SP, 1)
    out = pl.pallas_call(
        _tc_body,
        out_shape=jax.ShapeDtypeStruct((SP, D), jnp.float32),
    )(sums_p, cnts_p, W, b.reshape(1, D))
    return out[:S]


# speedup vs baseline: 9.0850x; 1.5761x over previous
import functools

import jax
import jax.numpy as jnp
from jax import lax
from jax.experimental import pallas as pl
from jax.experimental.pallas import tpu as pltpu
from jax.experimental.pallas import tpu_sc as plsc

N = 320000
D = 128
S = 10000
SP = 10240
NC, NS = 2, 16
NW = NC * NS
CHUNK = 128
TOT_CHUNKS = N // CHUNK
ITERS = -(-TOT_CHUNKS // NW)
ITERS_PAD = ITERS + (ITERS % 2)
SEG_PER_TILE = SP // NS


def _sc_body(x_hbm, idx_hbm, zacc_hbm, zcnt_hbm, ones_hbm,
             sums_hbm, cnts_hbm,
             acc_sh, cnt_sh, chunk_v, idx_v, ones_v, cstage_v, csem, isem):
    c = lax.axis_index("c")
    s = lax.axis_index("s")
    w = c * NS + s

    pltpu.sync_copy(zacc_hbm, chunk_v.at[0])
    pltpu.sync_copy(zcnt_hbm, cstage_v)
    for k in range(SEG_PER_TILE // CHUNK):
        off = s * SEG_PER_TILE + k * CHUNK
        pltpu.sync_copy(chunk_v.at[0], acc_sh.at[pl.ds(off, CHUNK)])
    pltpu.sync_copy(cstage_v, cnt_sh.at[pl.ds(s * SEG_PER_TILE, SEG_PER_TILE)])
    pltpu.sync_copy(ones_hbm, ones_v)
    plsc.subcore_barrier()

    def start_load(it, b):
        cid = it * NW + w

        @pl.when(cid < TOT_CHUNKS)
        def _():
            pltpu.async_copy(x_hbm.at[cid], chunk_v.at[b], csem.at[b])
            pltpu.async_copy(idx_hbm.at[cid], idx_v.at[b], isem.at[b])

    start_load(0, 0)

    @pl.loop(0, ITERS_PAD, step=2)
    def _loop(i):
        for b in range(2):
            it = i + b
            cid = it * NW + w
            start_load(it + 1, 1 - b)

            @pl.when(cid < TOT_CHUNKS)
            def _():
                pltpu.make_async_copy(x_hbm.at[cid], chunk_v.at[b],
                                      csem.at[b]).wait()
                pltpu.make_async_copy(idx_hbm.at[cid], idx_v.at[b],
                                      isem.at[b]).wait()
                pltpu.sync_copy(chunk_v.at[b], acc_sh.at[idx_v.at[b, 0]],
                                add=True)
                pltpu.sync_copy(ones_v, cnt_sh.at[idx_v.at[b, 0]], add=True)

    plsc.subcore_barrier()

    for k in range(SEG_PER_TILE // CHUNK):
        off = s * SEG_PER_TILE + k * CHUNK
        pltpu.sync_copy(acc_sh.at[pl.ds(off, CHUNK)], chunk_v.at[0])
        pltpu.sync_copy(chunk_v.at[0], sums_hbm.at[c, pl.ds(off, CHUNK)])
    pltpu.sync_copy(cnt_sh.at[pl.ds(s * SEG_PER_TILE, SEG_PER_TILE)], cstage_v)
    pltpu.sync_copy(cstage_v, cnts_hbm.at[c, pl.ds(s * SEG_PER_TILE, SEG_PER_TILE)])


_sc_segment_sum = functools.partial(
    pl.kernel,
    out_type=(
        jax.ShapeDtypeStruct((NC, SP, D), jnp.float32),
        jax.ShapeDtypeStruct((NC, SP), jnp.float32),
    ),
    mesh=plsc.VectorSubcoreMesh(core_axis_name="c", subcore_axis_name="s"),
    scratch_types=[
        pltpu.VMEM_SHARED((SP, D), jnp.float32),
        pltpu.VMEM_SHARED((SP,), jnp.float32),
        pltpu.VMEM((2, CHUNK, D), jnp.float32),
        pltpu.VMEM((2, 1, CHUNK), jnp.int32),
        pltpu.VMEM((CHUNK,), jnp.float32),
        pltpu.VMEM((SEG_PER_TILE,), jnp.float32),
        pltpu.SemaphoreType.DMA((2,)),
        pltpu.SemaphoreType.DMA((2,)),
    ],
)(_sc_body)


def _tc_body(sums_ref, cnt_ref, w_ref, b_ref, o_ref):
    sums = sums_ref[0] + sums_ref[1]
    cnt = cnt_ref[0] + cnt_ref[1]
    mean = sums / jnp.maximum(cnt, 1.0)
    out = lax.dot_general(mean, w_ref[...], (((1,), (1,)), ((), ())),
                          preferred_element_type=jnp.float32)
    o_ref[...] = out + jnp.where(cnt > 0.0, b_ref[...], 0.0)


def kernel(x, batch, W, b):
    x3 = x.reshape(TOT_CHUNKS, CHUNK, D)
    batch3 = batch.astype(jnp.int32).reshape(TOT_CHUNKS, 1, CHUNK)
    zacc = jnp.zeros((CHUNK, D), jnp.float32)
    zcnt = jnp.zeros((SEG_PER_TILE,), jnp.float32)
    ones = jnp.ones((CHUNK,), jnp.float32)
    sums_p, cnts_p = _sc_segment_sum(x3, batch3, zacc, zcnt, ones)
    cnts_p = cnts_p.reshape(NC, SP, 1)
    out = pl.pallas_call(
        _tc_body,
        out_shape=jax.ShapeDtypeStruct((SP, D), jnp.float32),
    )(sums_p, cnts_p, W, b.reshape(1, D))
    return out[:S]
